# Initial kernel scaffold; baseline (speedup 1.0000x reference)
import jax
import jax.numpy as jnp
from jax.experimental import pallas as pl


def kernel(x, token_table, pos_table):
    raise NotImplementedError("write your pallas kernel here")


# speedup vs baseline: 1.2751x; 1.2751x over previous
import functools

import jax
import jax.numpy as jnp
from jax import lax
from jax.experimental import pallas as pl
from jax.experimental.pallas import tpu as pltpu
from jax.experimental.pallas import tpu_sc as plsc

VOCAB = 100000
EMBED_DIM = 128
BLOCK = 2048
LANES = 16


def _build(B, T, D):
    info = plsc.get_sparse_core_info()
    NC, NS = info.num_cores, info.num_subcores
    NW = NC * NS
    n_tok = B * T
    b_per_w = n_tok // NW
    n_chunks = b_per_w // 128
    vregs_per_row = D // LANES

    mesh = plsc.VectorSubcoreMesh(core_axis_name="c", subcore_axis_name="s")

    @functools.partial(
        pl.kernel,
        mesh=mesh,
        out_type=jax.ShapeDtypeStruct((n_tok, D), jnp.float32),
        scratch_types=[
            pltpu.VMEM((n_chunks, 128), jnp.int32),
            pltpu.VMEM((b_per_w, D), jnp.float32),
            pltpu.VMEM((b_per_w, D), jnp.float32),
            pltpu.SemaphoreType.DMA,
        ],
    )
    def emb(x2d_hbm, table_hbm, pos_hbm, out_hbm, idx_v, rows_v, pos_v, sem):
        wid = lax.axis_index("s") * NC + lax.axis_index("c")
        base = wid * b_per_w
        pos_base = lax.rem(base, T)

        pltpu.sync_copy(x2d_hbm.at[pl.ds(wid * n_chunks, n_chunks)], idx_v)

        cps = [
            pltpu.async_copy(
                table_hbm.at[idx_v.at[c]],
                rows_v.at[pl.ds(c * 128, 128)],
                sem,
            )
            for c in range(n_chunks)
        ]
        pltpu.sync_copy(pos_hbm.at[pl.ds(pos_base, b_per_w)], pos_v)
        for cp in cps:
            cp.wait()

        def row_body(r, carry):
            for j in range(vregs_per_row):
                s = pl.ds(j * LANES, LANES)
                rows_v[r, s] = rows_v[r, s] + pos_v[r, s]
            return carry

        lax.fori_loop(0, b_per_w, row_body, 0)

        pltpu.sync_copy(rows_v, out_hbm.at[pl.ds(base, b_per_w)])

    return emb


def kernel(x, token_table, pos_table):
    B, T = x.shape
    D = token_table.shape[1]
    x2d = jnp.reshape(x.astype(jnp.int32), (-1, 128))
    out = _build(B, T, D)(x2d, token_table, pos_table)
    return jnp.reshape(out, (B, T, D))


# speedup vs baseline: 1.2990x; 1.0187x over previous
import functools

import jax
import jax.numpy as jnp
from jax import lax
from jax.experimental import pallas as pl
from jax.experimental.pallas import tpu as pltpu
from jax.experimental.pallas import tpu_sc as plsc

LANES = 16
CHUNK = 64


def _build(B, T, D):
    info = plsc.get_sparse_core_info()
    NC, NS = info.num_cores, info.num_subcores
    NW = NC * NS
    n_tok = B * T
    b_per_w = n_tok // NW
    w_per_row = T // b_per_w
    n_chunks = b_per_w // CHUNK
    vregs_per_row = D // LANES

    mesh = plsc.VectorSubcoreMesh(core_axis_name="c", subcore_axis_name="s")

    @functools.partial(
        pl.kernel,
        mesh=mesh,
        out_type=jax.ShapeDtypeStruct((B, T, D), jnp.float32),
        scratch_types=[
            pltpu.VMEM((b_per_w,), jnp.int32),
            pltpu.VMEM((b_per_w, D), jnp.float32),
            pltpu.VMEM((b_per_w, D), jnp.float32),
        ]
        + [pltpu.SemaphoreType.DMA] * n_chunks
        + [pltpu.SemaphoreType.DMA],
    )
    def emb(x_hbm, table_hbm, pos_hbm, out_hbm, idx_v, rows_v, pos_v, *sems):
        gsems, osem = sems[:n_chunks], sems[n_chunks]
        wid = lax.axis_index("s") * NC + lax.axis_index("c")
        b = lax.div(wid, w_per_row)
        col = lax.rem(wid, w_per_row) * b_per_w

        pltpu.sync_copy(x_hbm.at[b, pl.ds(col, b_per_w)], idx_v)

        gathers = [
            pltpu.async_copy(
                table_hbm.at[idx_v.at[pl.ds(c * CHUNK, CHUNK)]],
                rows_v.at[pl.ds(c * CHUNK, CHUNK)],
                gsems[c],
            )
            for c in range(n_chunks)
        ]
        pltpu.sync_copy(pos_hbm.at[pl.ds(col, b_per_w)], pos_v)

        outs = []
        for c in range(n_chunks):
            gathers[c].wait()

            def row_body(r, carry):
                for j in range(vregs_per_row):
                    s = pl.ds(j * LANES, LANES)
                    rows_v[r, s] = rows_v[r, s] + pos_v[r, s]
                return carry

            lax.fori_loop(c * CHUNK, (c + 1) * CHUNK, row_body, 0)
            outs.append(
                pltpu.async_copy(
                    rows_v.at[pl.ds(c * CHUNK, CHUNK)],
                    out_hbm.at[b, pl.ds(col + c * CHUNK, CHUNK)],
                    osem,
                )
            )
        for cp in outs:
            cp.wait()

    return emb


def kernel(x, token_table, pos_table):
    B, T = x.shape
    D = token_table.shape[1]
    return _build(B, T, D)(x.astype(jnp.int32), token_table, pos_table)


# speedup vs baseline: 1.3514x; 1.0404x over previous
import functools

import jax
import jax.numpy as jnp
from jax import lax
from jax.experimental import pallas as pl
from jax.experimental.pallas import tpu as pltpu
from jax.experimental.pallas import tpu_sc as plsc

LANES = 16


def _build(B, T, D):
    info = plsc.get_sparse_core_info()
    NC, NS = info.num_cores, info.num_subcores
    NW = NC * NS
    t_per_w = T // NW
    vregs_per_row = D // LANES

    mesh = plsc.VectorSubcoreMesh(core_axis_name="c", subcore_axis_name="s")

    @functools.partial(
        pl.kernel,
        mesh=mesh,
        out_type=jax.ShapeDtypeStruct((B, T, D), jnp.float32),
        scratch_types=[
            pltpu.VMEM((B * t_per_w,), jnp.int32),
            pltpu.VMEM((B * t_per_w, D), jnp.float32),
            pltpu.VMEM((t_per_w, D), jnp.float32),
        ]
        + [pltpu.SemaphoreType.DMA] * B
        + [pltpu.SemaphoreType.DMA, pltpu.SemaphoreType.DMA],
    )
    def emb(x_hbm, table_hbm, pos_hbm, out_hbm, idx_v, rows_v, pos_v, *sems):
        gsems, osem, isem = sems[:B], sems[B], sems[B + 1]
        wid = lax.axis_index("s") * NC + lax.axis_index("c")
        col = wid * t_per_w

        idx_cps = [
            pltpu.async_copy(
                x_hbm.at[b, pl.ds(col, t_per_w)],
                idx_v.at[pl.ds(b * t_per_w, t_per_w)],
                isem,
            )
            for b in range(B)
        ]
        for cp in idx_cps:
            cp.wait()

        gathers = [
            pltpu.async_copy(
                table_hbm.at[idx_v.at[pl.ds(b * t_per_w, t_per_w)]],
                rows_v.at[pl.ds(b * t_per_w, t_per_w)],
                gsems[b],
            )
            for b in range(B)
        ]
        pltpu.sync_copy(pos_hbm.at[pl.ds(col, t_per_w)], pos_v)

        outs = []
        for b in range(B):
            gathers[b].wait()

            def row_body(r, carry, base=b * t_per_w):
                for j in range(vregs_per_row):
                    s = pl.ds(j * LANES, LANES)
                    rows_v[base + r, s] = rows_v[base + r, s] + pos_v[r, s]
                return carry

            lax.fori_loop(0, t_per_w, row_body, 0)
            outs.append(
                pltpu.async_copy(
                    rows_v.at[pl.ds(b * t_per_w, t_per_w)],
                    out_hbm.at[b, pl.ds(col, t_per_w)],
                    osem,
                )
            )
        for cp in outs:
            cp.wait()

    return emb


def kernel(x, token_table, pos_table):
    B, T = x.shape
    D = token_table.shape[1]
    return _build(B, T, D)(x.astype(jnp.int32), token_table, pos_table)


# speedup vs baseline: 1.3804x; 1.0215x over previous
import functools

import jax
import jax.numpy as jnp
from jax import lax
from jax.experimental import pallas as pl
from jax.experimental.pallas import tpu as pltpu
from jax.experimental.pallas import tpu_sc as plsc

LANES = 16
HALVES = 2


def _build(B, T, D):
    info = plsc.get_sparse_core_info()
    NC, NS = info.num_cores, info.num_subcores
    NW = NC * NS
    t_per_w = T // NW
    t_half = t_per_w // HALVES
    vregs_per_row = D // LANES

    mesh = plsc.VectorSubcoreMesh(core_axis_name="c", subcore_axis_name="s")

    @functools.partial(
        pl.kernel,
        mesh=mesh,
        out_type=jax.ShapeDtypeStruct((B, T, D), jnp.float32),
        scratch_types=[
            pltpu.VMEM((B * t_per_w,), jnp.int32),
            pltpu.VMEM((B * t_per_w, D), jnp.float32),
            pltpu.VMEM((t_per_w, D), jnp.float32),
        ]
        + [pltpu.SemaphoreType.DMA] * HALVES
        + [pltpu.SemaphoreType.DMA, pltpu.SemaphoreType.DMA],
    )
    def emb(x_hbm, table_hbm, pos_hbm, out_hbm, idx_v, rows_v, pos_v, *sems):
        hsems, osem, isem = sems[:HALVES], sems[HALVES], sems[HALVES + 1]
        wid = lax.axis_index("s") * NC + lax.axis_index("c")
        col = wid * t_per_w

        idx_cps = [
            pltpu.async_copy(
                x_hbm.at[b, pl.ds(col, t_per_w)],
                idx_v.at[pl.ds(b * t_per_w, t_per_w)],
                isem,
            )
            for b in range(B)
        ]
        for cp in idx_cps:
            cp.wait()

        gathers = [
            [
                pltpu.async_copy(
                    table_hbm.at[idx_v.at[pl.ds(b * t_per_w + h * t_half, t_half)]],
                    rows_v.at[pl.ds(b * t_per_w + h * t_half, t_half)],
                    hsems[h],
                )
                for b in range(B)
            ]
            for h in range(HALVES)
        ]
        pltpu.sync_copy(pos_hbm.at[pl.ds(col, t_per_w)], pos_v)

        outs = []
        for h in range(HALVES):
            for cp in gathers[h]:
                cp.wait()

            def row_body(r, carry):
                pos_regs = [
                    pos_v[r, pl.ds(j * LANES, LANES)] for j in range(vregs_per_row)
                ]
                for b in range(B):
                    base = b * t_per_w
                    for j in range(vregs_per_row):
                        s = pl.ds(j * LANES, LANES)
                        rows_v[base + r, s] = rows_v[base + r, s] + pos_regs[j]
                return carry

            lax.fori_loop(h * t_half, (h + 1) * t_half, row_body, 0)
            outs.extend(
                pltpu.async_copy(
                    rows_v.at[pl.ds(b * t_per_w + h * t_half, t_half)],
                    out_hbm.at[b, pl.ds(col + h * t_half, t_half)],
                    osem,
                )
                for b in range(B)
            )
        for cp in outs:
            cp.wait()

    return emb


def kernel(x, token_table, pos_table):
    B, T = x.shape
    D = token_table.shape[1]
    return _build(B, T, D)(x.astype(jnp.int32), token_table, pos_table)
